# baseline (speedup 1.0000x reference)
import functools

import jax
import jax.numpy as jnp
from jax import lax
from jax.experimental import pallas as pl
from jax.experimental.pallas import tpu as pltpu
from jax.experimental.pallas import tpu_sc as plsc

_NC = 2
_NS = 16
_NW = _NC * _NS
_CHUNK = 32


_SCS_CHUNK = 256


@functools.partial(jax.jit, static_argnums=(1, 2, 3))
def _broadcast_rows(table_flat, batch, max_len, d):
    rows_per_w = max_len // _NC
    n_chunks = rows_per_w // _SCS_CHUNK
    chunk_elems = _SCS_CHUNK * d

    def body(table_hbm, out_hbm, buf0, buf1, in_sem, out_sem):
        wid = lax.axis_index("c")
        base = wid * rows_per_w
        bufs = (buf0, buf1)

        def in_copy(i):
            return pltpu.make_async_copy(
                table_hbm.at[pl.ds((base + i * _SCS_CHUNK + 1) * d, chunk_elems)],
                bufs[i % 2], in_sem)

        def out_copies(i):
            row0 = base + i * _SCS_CHUNK
            return [
                pltpu.make_async_copy(
                    bufs[i % 2],
                    out_hbm.at[pl.ds((b * max_len + row0) * d, chunk_elems)],
                    out_sem)
                for b in range(batch)
            ]

        pending_in = in_copy(0)
        pending_in.start()
        pending_out = []
        for i in range(n_chunks):
            pending_in.wait()
            outs = out_copies(i)
            for cp in outs:
                cp.start()
            if i + 1 < n_chunks:
                for cp in pending_out:
                    cp.wait()
                pending_out = outs
                pending_in = in_copy(i + 1)
                pending_in.start()
            else:
                for cp in pending_out:
                    cp.wait()
                for cp in outs:
                    cp.wait()

    return pl.kernel(
        body,
        out_type=jax.ShapeDtypeStruct((batch * max_len * d,), table_flat.dtype),
        mesh=plsc.ScalarSubcoreMesh(axis_name="c", num_cores=_NC),
        scratch_types=[
            pltpu.VMEM_SHARED((chunk_elems,), table_flat.dtype),
            pltpu.VMEM_SHARED((chunk_elems,), table_flat.dtype),
            pltpu.SemaphoreType.DMA,
            pltpu.SemaphoreType.DMA,
        ],
    )(table_flat)


@functools.partial(jax.jit, static_argnums=(1, 2, 3))
def _broadcast_rows_tec(table_flat, batch, max_len, d):
    rows_per_w = max_len // _NW
    n_chunks = rows_per_w // _CHUNK
    chunk_elems = _CHUNK * d

    def body(table_hbm, out_hbm, buf0, buf1, in_sem, out_sem):
        c = lax.axis_index("c")
        s = lax.axis_index("s")
        wid = s * _NC + c
        base = wid * rows_per_w
        bufs = (buf0, buf1)

        def in_copy(i):
            return pltpu.make_async_copy(
                table_hbm.at[pl.ds((base + i * _CHUNK + 1) * d, chunk_elems)],
                bufs[i % 2], in_sem)

        def out_copies(i):
            row0 = base + i * _CHUNK
            return [
                pltpu.make_async_copy(
                    bufs[i % 2],
                    out_hbm.at[pl.ds((b * max_len + row0) * d, chunk_elems)],
                    out_sem)
                for b in range(batch)
            ]

        pending_in = in_copy(0)
        pending_in.start()
        pending_out = []
        for i in range(n_chunks):
            pending_in.wait()
            outs = out_copies(i)
            for cp in outs:
                cp.start()
            if i + 1 < n_chunks:
                for cp in pending_out:
                    cp.wait()
                pending_out = outs
                pending_in = in_copy(i + 1)
                pending_in.start()
            else:
                for cp in pending_out:
                    cp.wait()
                for cp in outs:
                    cp.wait()

    return pl.kernel(
        body,
        out_type=jax.ShapeDtypeStruct((batch * max_len * d,), table_flat.dtype),
        mesh=plsc.VectorSubcoreMesh(
            core_axis_name="c", subcore_axis_name="s",
            num_cores=_NC, num_subcores=_NS),
        scratch_types=[
            pltpu.VMEM((chunk_elems,), table_flat.dtype),
            pltpu.VMEM((chunk_elems,), table_flat.dtype),
            pltpu.SemaphoreType.DMA,
            pltpu.SemaphoreType.DMA,
        ],
    )(table_flat)


def kernel(x, emb_table):
    batch, max_len = x.shape
    d = emb_table.shape[1]
    flat = _broadcast_rows(emb_table.reshape(-1), batch, max_len, d)
    return flat.reshape(batch, max_len, d)


# speedup vs baseline: 3.3445x; 3.3445x over previous
import functools

import jax
import jax.numpy as jnp
from jax import lax
from jax.experimental import pallas as pl
from jax.experimental.pallas import tpu as pltpu
from jax.experimental.pallas import tpu_sc as plsc

_NC = 2
_NS = 16
_NW = _NC * _NS
_LANES = 16
_CHUNK = 32


@functools.partial(jax.jit, static_argnums=(1, 2, 3))
def _broadcast_rows(emb_table, batch, max_len, d):
    rows_per_w = max_len // _NW
    n_chunks = rows_per_w // _CHUNK

    def body(table_hbm, out_hbm, buf0, buf1, idx0, idx1, in_sem, out_sem):
        c = lax.axis_index("c")
        s = lax.axis_index("s")
        wid = s * _NC + c
        base = wid * rows_per_w
        bufs = (buf0, buf1)
        idxs = (idx0, idx1)
        lane = lax.iota(jnp.int32, _LANES)

        def in_copy(i):
            row0 = base + i * _CHUNK + 1
            idx = idxs[i % 2]
            for j in range(_CHUNK // _LANES):
                idx[pl.ds(j * _LANES, _LANES)] = row0 + j * _LANES + lane
            return pltpu.make_async_copy(
                table_hbm.at[idx], bufs[i % 2], in_sem)

        def out_copies(i):
            row0 = base + i * _CHUNK
            return [
                pltpu.make_async_copy(
                    bufs[i % 2],
                    out_hbm.at[b, pl.ds(row0, _CHUNK)],
                    out_sem)
                for b in range(batch)
            ]

        pending_in = in_copy(0)
        pending_in.start()
        pending_out = []
        for i in range(n_chunks):
            pending_in.wait()
            outs = out_copies(i)
            for cp in outs:
                cp.start()
            if i + 1 < n_chunks:
                for cp in pending_out:
                    cp.wait()
                pending_out = outs
                pending_in = in_copy(i + 1)
                pending_in.start()
            else:
                for cp in pending_out:
                    cp.wait()
                for cp in outs:
                    cp.wait()

    return pl.kernel(
        body,
        out_type=jax.ShapeDtypeStruct((batch, max_len, d), emb_table.dtype),
        mesh=plsc.VectorSubcoreMesh(
            core_axis_name="c", subcore_axis_name="s",
            num_cores=_NC, num_subcores=_NS),
        scratch_types=[
            pltpu.VMEM((_CHUNK, d), emb_table.dtype),
            pltpu.VMEM((_CHUNK, d), emb_table.dtype),
            pltpu.VMEM((_CHUNK,), jnp.int32),
            pltpu.VMEM((_CHUNK,), jnp.int32),
            pltpu.SemaphoreType.DMA,
            pltpu.SemaphoreType.DMA,
        ],
    )(emb_table)


def kernel(x, emb_table):
    batch, max_len = x.shape
    d = emb_table.shape[1]
    return _broadcast_rows(emb_table, batch, max_len, d)


# speedup vs baseline: 3.3589x; 1.0043x over previous
import functools

import jax
import jax.numpy as jnp
from jax import lax
from jax.experimental import pallas as pl
from jax.experimental.pallas import tpu as pltpu
from jax.experimental.pallas import tpu_sc as plsc

_NC = 2
_NS = 16
_NW = _NC * _NS
_LANES = 16
_CHUNK = 32
_NBUF = 3


@functools.partial(jax.jit, static_argnums=(1, 2, 3))
def _broadcast_rows(emb_table, batch, max_len, d):
    rows_per_w = max_len // _NW
    n_chunks = rows_per_w // _CHUNK

    def body(table_hbm, out_hbm, *refs):
        bufs = refs[:_NBUF]
        idxs = refs[_NBUF:2 * _NBUF]
        in_sem, out_sem = refs[2 * _NBUF], refs[2 * _NBUF + 1]
        c = lax.axis_index("c")
        s = lax.axis_index("s")
        wid = s * _NC + c
        base = wid * rows_per_w
        lane = lax.iota(jnp.int32, _LANES)

        def in_copy(i):
            row0 = base + i * _CHUNK + 1
            idx = idxs[i % _NBUF]
            for j in range(_CHUNK // _LANES):
                idx[pl.ds(j * _LANES, _LANES)] = row0 + j * _LANES + lane
            return pltpu.make_async_copy(
                table_hbm.at[idx], bufs[i % _NBUF], in_sem)

        def out_copies(i):
            row0 = base + i * _CHUNK
            return [
                pltpu.make_async_copy(
                    bufs[i % _NBUF],
                    out_hbm.at[b, pl.ds(row0, _CHUNK)],
                    out_sem)
                for b in range(batch)
            ]

        in_h = [None] * n_chunks
        out_h = [None] * n_chunks
        in_h[0] = in_copy(0)
        in_h[0].start()
        drained = 0
        for i in range(n_chunks):
            in_h[i].wait()
            out_h[i] = out_copies(i)
            for cp in out_h[i]:
                cp.start()
            if i + 1 < n_chunks:
                if i + 1 >= _NBUF:
                    for cp in out_h[i + 1 - _NBUF]:
                        cp.wait()
                    drained = i + 2 - _NBUF
                in_h[i + 1] = in_copy(i + 1)
                in_h[i + 1].start()
        for i in range(drained, n_chunks):
            for cp in out_h[i]:
                cp.wait()

    return pl.kernel(
        body,
        out_type=jax.ShapeDtypeStruct((batch, max_len, d), emb_table.dtype),
        mesh=plsc.VectorSubcoreMesh(
            core_axis_name="c", subcore_axis_name="s",
            num_cores=_NC, num_subcores=_NS),
        scratch_types=(
            [pltpu.VMEM((_CHUNK, d), emb_table.dtype) for _ in range(_NBUF)]
            + [pltpu.VMEM((_CHUNK,), jnp.int32) for _ in range(_NBUF)]
            + [pltpu.SemaphoreType.DMA, pltpu.SemaphoreType.DMA]
        ),
    )(emb_table)


def kernel(x, emb_table):
    batch, max_len = x.shape
    d = emb_table.shape[1]
    return _broadcast_rows(emb_table, batch, max_len, d)
